# baseline (speedup 1.0000x reference)
import functools

import jax
import jax.numpy as jnp
from jax import lax
from jax.experimental import pallas as pl
from jax.experimental.pallas import tpu as pltpu
from jax.experimental.pallas import tpu_sc as plsc

_N, _C, _H, _W = 2, 490, 50, 50
_PH, _PW = 7, 7
_NBINS = _PH * _PW
_COUT = _C // _NBINS
_SCALE = 0.0625
_GRID = 2
_TERMS = _NBINS * _GRID * _GRID * 4
_NROIS = 512
_VROWS = _N * _NBINS * _H * _W
_LANES = 16

_NCORES, _NSUBCORES = 2, 16
_NTILES = _NCORES * _NSUBCORES
_ROIS_PER_TILE = _NROIS // _NTILES
_CHUNK = 112
_NCHUNKS = _TERMS // _CHUNK


def _table_kernel(inp_ref, out_ref):
    b = pl.program_id(1)
    x = inp_ref[0, :, b, :]
    eye = (
        lax.broadcasted_iota(jnp.int32, (_COUT, _LANES), 0)
        == lax.broadcasted_iota(jnp.int32, (_COUT, _LANES), 1)
    ).astype(jnp.float32)
    out_ref[0, 0] = lax.dot_general(
        x, eye, (((0,), (0,)), ((), ())),
        preferred_element_type=jnp.float32,
        precision=lax.Precision.HIGHEST,
    )


def _build_table(input):
    x = input.reshape(_N, _COUT, _NBINS, _H * _W)
    out = pl.pallas_call(
        _table_kernel,
        grid=(_N, _NBINS),
        in_specs=[pl.BlockSpec((1, _COUT, _NBINS, _H * _W), lambda n, b: (n, 0, 0, 0))],
        out_specs=pl.BlockSpec((1, 1, _H * _W, _LANES), lambda n, b: (n, b, 0, 0)),
        out_shape=jax.ShapeDtypeStruct((_N, _NBINS, _H * _W, _LANES), jnp.float32),
    )(x)
    return out.reshape(_VROWS, _LANES)


def _terms_kernel(rois_ref, idx_ref, w_ref):
    shp = (_NROIS, _TERMS)
    t = lax.broadcasted_iota(jnp.int32, shp, 1)
    b = t // 16
    j = t - 16 * b
    ph = b // _PW
    pw = b - _PW * ph
    iy = j // 8
    ix = (j - 8 * iy) // 4
    c = j - 8 * iy - 4 * ix

    n = rois_ref[:, 0:1].astype(jnp.int32)
    sw = rois_ref[:, 1:2] * _SCALE - 0.5
    sh = rois_ref[:, 2:3] * _SCALE - 0.5
    ew = rois_ref[:, 3:4] * _SCALE - 0.5
    eh = rois_ref[:, 4:5] * _SCALE - 0.5
    bh = (eh - sh) * (1.0 / _PH)
    bw = (ew - sw) * (1.0 / _PW)

    y = sh + ph.astype(jnp.float32) * bh + (iy.astype(jnp.float32) + 0.5) * bh * (1.0 / _GRID)
    x = sw + pw.astype(jnp.float32) * bw + (ix.astype(jnp.float32) + 0.5) * bw * (1.0 / _GRID)
    valid = (y >= -1.0) & (y <= float(_H)) & (x >= -1.0) & (x <= float(_W))

    yc = jnp.maximum(y, 0.0)
    y_low = jnp.floor(yc).astype(jnp.int32)
    y_edge = y_low >= _H - 1
    y_high = jnp.where(y_edge, _H - 1, y_low + 1)
    y_low = jnp.where(y_edge, _H - 1, y_low)
    yc = jnp.where(y_edge, y_low.astype(jnp.float32), yc)
    ly = yc - y_low.astype(jnp.float32)
    hy = 1.0 - ly

    xc = jnp.maximum(x, 0.0)
    x_low = jnp.floor(xc).astype(jnp.int32)
    x_edge = x_low >= _W - 1
    x_high = jnp.where(x_edge, _W - 1, x_low + 1)
    x_low = jnp.where(x_edge, _W - 1, x_low)
    xc = jnp.where(x_edge, x_low.astype(jnp.float32), xc)
    lx = xc - x_low.astype(jnp.float32)
    hx = 1.0 - lx

    yp = jnp.where(c >= 2, y_high, y_low)
    xp = jnp.where(c % 2 == 1, x_high, x_low)
    wy = jnp.where(c >= 2, ly, hy)
    wx = jnp.where(c % 2 == 1, lx, hx)
    w = jnp.where(valid, wy * wx * (1.0 / (_GRID * _GRID)), 0.0)

    row = ((n * _NBINS + b) * _H + yp) * _W + xp
    row = jnp.clip(row, 0, _VROWS - 1)
    idx_ref[...] = row
    w_ref[...] = w


def _compute_terms(rois):
    return pl.pallas_call(
        _terms_kernel,
        out_shape=(
            jax.ShapeDtypeStruct((_NROIS, _TERMS), jnp.int32),
            jax.ShapeDtypeStruct((_NROIS, _TERMS), jnp.float32),
        ),
    )(rois)


def _sc_body(table_hbm, idx_hbm, w_hbm, out_hbm, idx_v, w_v, g_v, out_v, sem):
    wid = lax.axis_index("s") * _NCORES + lax.axis_index("c")
    base = wid * _ROIS_PER_TILE
    pltpu.sync_copy(idx_hbm.at[pl.ds(base, _ROIS_PER_TILE)], idx_v)
    pltpu.sync_copy(w_hbm.at[pl.ds(base, _ROIS_PER_TILE)], w_v)

    def per_roi(r, carry):
        copies = [
            pltpu.async_copy(
                table_hbm.at[idx_v.at[r, j]],
                g_v.at[pl.ds(j * _CHUNK, _CHUNK)],
                sem,
            )
            for j in range(_NCHUNKS)
        ]
        for cp in copies:
            cp.wait()
        for b in range(_NBINS):
            wvec = w_v[r, pl.ds(b * 16, 16)]
            acc = wvec[0] * g_v[b * 16, :]
            for j in range(1, 16):
                acc = acc + wvec[j] * g_v[b * 16 + j, :]
            out_v[b, :] = acc
        pltpu.sync_copy(out_v, out_hbm.at[base + r])
        return carry

    lax.fori_loop(0, _ROIS_PER_TILE, per_roi, 0)


@functools.cache
def _sc_gather():
    return pl.kernel(
        _sc_body,
        out_type=jax.ShapeDtypeStruct((_NROIS, _NBINS, _LANES), jnp.float32),
        mesh=plsc.VectorSubcoreMesh(
            core_axis_name="c", subcore_axis_name="s",
            num_cores=_NCORES, num_subcores=_NSUBCORES,
        ),
        scratch_types=[
            pltpu.VMEM((_ROIS_PER_TILE, _NCHUNKS, _CHUNK), jnp.int32),
            pltpu.VMEM((_ROIS_PER_TILE, _TERMS), jnp.float32),
            pltpu.VMEM((_TERMS, _LANES), jnp.float32),
            pltpu.VMEM((_NBINS, _LANES), jnp.float32),
            pltpu.SemaphoreType.DMA,
        ],
        compiler_params=pltpu.CompilerParams(use_tc_tiling_on_sc=False),
    )


def kernel(input, rois):
    table = _build_table(input)
    idx, w = _compute_terms(rois)
    idx = idx.reshape(_NROIS, _NCHUNKS, _CHUNK)

    out = _sc_gather()(table, idx, w)
    return out[:, :, :_COUT].transpose(0, 2, 1).reshape(_NROIS, _COUT, _PH, _PW)


# speedup vs baseline: 1.9028x; 1.9028x over previous
import functools

import jax
import jax.numpy as jnp
from jax import lax
from jax.experimental import pallas as pl
from jax.experimental.pallas import tpu as pltpu
from jax.experimental.pallas import tpu_sc as plsc

_N, _C, _H, _W = 2, 490, 50, 50
_PH, _PW = 7, 7
_NBINS = _PH * _PW
_COUT = _C // _NBINS
_SCALE = 0.0625
_GRID = 2
_TERMS = _NBINS * _GRID * _GRID * 4
_NROIS = 512
_VROWS = _N * _NBINS * _H * _W
_LANES = 16

_NCORES, _NSUBCORES = 2, 16
_NTILES = _NCORES * _NSUBCORES
_ROIS_PER_TILE = _NROIS // _NTILES
_CHUNK = 112
_NCHUNKS = _TERMS // _CHUNK


_HW = _H * _W
_NPAIRS = _N * _NBINS
_KSTEPS = _HW // _LANES
_KREM = _HW - _KSTEPS * _LANES


def _table_body(inp_hbm, out_hbm, chan_v, ttile_v, sem):
    wid = lax.axis_index("s") * _NCORES + lax.axis_index("c")
    lanes = lax.iota(jnp.int32, _LANES)

    def per_pair(p, carry):
        n = p // _NBINS
        b = p - n * _NBINS
        pltpu.sync_copy(inp_hbm.at[n, :, b, :], chan_v)

        def per_k(k, carry2):
            src = pl.ds(k * _LANES, _LANES)
            tgt = (lanes + k * _LANES) * _LANES
            for ctop in range(_COUT):
                plsc.store_scatter(ttile_v, [tgt + ctop], chan_v[ctop, src])
            return carry2

        lax.fori_loop(0, _KSTEPS, per_k, 0)
        rem_mask = lanes >= _LANES - _KREM
        rem0 = _HW - _LANES
        tgt = (lanes + rem0) * _LANES
        for ctop in range(_COUT):
            v = chan_v[ctop, pl.ds(rem0, _LANES)]
            plsc.store_scatter(ttile_v, [tgt + ctop], v, mask=rem_mask)
        pltpu.sync_copy(ttile_v, out_hbm.at[pl.ds(p * (_HW * _LANES), _HW * _LANES)])
        return carry

    def per_slot(i, carry):
        per_pair(wid + i * _NTILES, carry)
        return carry

    lax.fori_loop(0, 3, per_slot, 0)

    @pl.when(wid + 3 * _NTILES < _NPAIRS)
    def _():
        per_pair(wid + 3 * _NTILES, 0)


@functools.cache
def _table_builder():
    return pl.kernel(
        _table_body,
        out_type=jax.ShapeDtypeStruct((_VROWS * _LANES,), jnp.float32),
        mesh=plsc.VectorSubcoreMesh(
            core_axis_name="c", subcore_axis_name="s",
            num_cores=_NCORES, num_subcores=_NSUBCORES,
        ),
        scratch_types=[
            pltpu.VMEM((_COUT, _HW), jnp.float32),
            pltpu.VMEM((_HW * _LANES,), jnp.float32),
            pltpu.SemaphoreType.DMA,
        ],
        compiler_params=pltpu.CompilerParams(
            use_tc_tiling_on_sc=False, needs_layout_passes=False
        ),
    )


def _build_table(input):
    x = input.reshape(_N, _COUT, _NBINS, _HW)
    return _table_builder()(x).reshape(_VROWS, _LANES)


def _terms_kernel(rois_ref, idx_ref, w_ref):
    shp = (_NROIS, _TERMS)
    t = lax.broadcasted_iota(jnp.int32, shp, 1)
    b = t // 16
    j = t - 16 * b
    ph = b // _PW
    pw = b - _PW * ph
    iy = j // 8
    ix = (j - 8 * iy) // 4
    c = j - 8 * iy - 4 * ix

    n = rois_ref[:, 0:1].astype(jnp.int32)
    sw = rois_ref[:, 1:2] * _SCALE - 0.5
    sh = rois_ref[:, 2:3] * _SCALE - 0.5
    ew = rois_ref[:, 3:4] * _SCALE - 0.5
    eh = rois_ref[:, 4:5] * _SCALE - 0.5
    bh = (eh - sh) * (1.0 / _PH)
    bw = (ew - sw) * (1.0 / _PW)

    y = sh + ph.astype(jnp.float32) * bh + (iy.astype(jnp.float32) + 0.5) * bh * (1.0 / _GRID)
    x = sw + pw.astype(jnp.float32) * bw + (ix.astype(jnp.float32) + 0.5) * bw * (1.0 / _GRID)
    valid = (y >= -1.0) & (y <= float(_H)) & (x >= -1.0) & (x <= float(_W))

    yc = jnp.maximum(y, 0.0)
    y_low = jnp.floor(yc).astype(jnp.int32)
    y_edge = y_low >= _H - 1
    y_high = jnp.where(y_edge, _H - 1, y_low + 1)
    y_low = jnp.where(y_edge, _H - 1, y_low)
    yc = jnp.where(y_edge, y_low.astype(jnp.float32), yc)
    ly = yc - y_low.astype(jnp.float32)
    hy = 1.0 - ly

    xc = jnp.maximum(x, 0.0)
    x_low = jnp.floor(xc).astype(jnp.int32)
    x_edge = x_low >= _W - 1
    x_high = jnp.where(x_edge, _W - 1, x_low + 1)
    x_low = jnp.where(x_edge, _W - 1, x_low)
    xc = jnp.where(x_edge, x_low.astype(jnp.float32), xc)
    lx = xc - x_low.astype(jnp.float32)
    hx = 1.0 - lx

    yp = jnp.where(c >= 2, y_high, y_low)
    xp = jnp.where(c % 2 == 1, x_high, x_low)
    wy = jnp.where(c >= 2, ly, hy)
    wx = jnp.where(c % 2 == 1, lx, hx)
    w = jnp.where(valid, wy * wx * (1.0 / (_GRID * _GRID)), 0.0)

    row = ((n * _NBINS + b) * _H + yp) * _W + xp
    row = jnp.clip(row, 0, _VROWS - 1)
    idx_ref[...] = row
    w_ref[...] = w


def _compute_terms(rois):
    return pl.pallas_call(
        _terms_kernel,
        out_shape=(
            jax.ShapeDtypeStruct((_NROIS, _TERMS), jnp.int32),
            jax.ShapeDtypeStruct((_NROIS, _TERMS), jnp.float32),
        ),
    )(rois)


def _sc_body(table_hbm, idx_hbm, w_hbm, out_hbm, idx_v, w_v, g_v, out_v, sem):
    wid = lax.axis_index("s") * _NCORES + lax.axis_index("c")
    base = wid * _ROIS_PER_TILE
    pltpu.sync_copy(idx_hbm.at[pl.ds(base, _ROIS_PER_TILE)], idx_v)
    pltpu.sync_copy(w_hbm.at[pl.ds(base, _ROIS_PER_TILE)], w_v)
    lanes = lax.iota(jnp.int32, _LANES)
    cmask = lanes < _COUT
    cidx = lanes * _NBINS

    def per_roi(r, carry):
        copies = [
            pltpu.async_copy(
                table_hbm.at[idx_v.at[r, j]],
                g_v.at[pl.ds(j * _CHUNK, _CHUNK)],
                sem,
            )
            for j in range(_NCHUNKS)
        ]
        for cp in copies:
            cp.wait()
        for b in range(_NBINS):
            wvec = w_v[r, pl.ds(b * 16, 16)]
            acc = wvec[0] * g_v[b * 16, :]
            for j in range(1, 16):
                acc = acc + wvec[j] * g_v[b * 16 + j, :]
            plsc.store_scatter(out_v, [cidx + b], acc, mask=cmask)
        pltpu.sync_copy(out_v, out_hbm.at[base + r])
        return carry

    lax.fori_loop(0, _ROIS_PER_TILE, per_roi, 0)


@functools.cache
def _sc_gather():
    return pl.kernel(
        _sc_body,
        out_type=jax.ShapeDtypeStruct((_NROIS, _C), jnp.float32),
        mesh=plsc.VectorSubcoreMesh(
            core_axis_name="c", subcore_axis_name="s",
            num_cores=_NCORES, num_subcores=_NSUBCORES,
        ),
        scratch_types=[
            pltpu.VMEM((_ROIS_PER_TILE, _NCHUNKS, _CHUNK), jnp.int32),
            pltpu.VMEM((_ROIS_PER_TILE, _TERMS), jnp.float32),
            pltpu.VMEM((_TERMS, _LANES), jnp.float32),
            pltpu.VMEM((_C,), jnp.float32),
            pltpu.SemaphoreType.DMA,
        ],
        compiler_params=pltpu.CompilerParams(
            use_tc_tiling_on_sc=False, needs_layout_passes=False
        ),
    )


def kernel(input, rois):
    table = _build_table(input)
    idx, w = _compute_terms(rois)
    idx = idx.reshape(_NROIS, _NCHUNKS, _CHUNK)

    out = _sc_gather()(table, idx, w)
    return out.reshape(_NROIS, _COUT, _PH, _PW)
